# baseline (speedup 1.0000x reference)
import functools

import jax
import jax.numpy as jnp
from jax import lax
from jax.experimental import pallas as pl
from jax.experimental.pallas import tpu as pltpu
from jax.experimental.pallas import tpu_sc as plsc

UPDATE_SIZE = 1024
BATCH = 64
NUM_UPD = 16
NUM_CH = 2
KEEP = NUM_UPD * UPDATE_SIZE
SNAP = (BATCH + NUM_UPD - 1) * UPDATE_SIZE
NBLK = BATCH + NUM_UPD - 1
OUT_LEN = BATCH * UPDATE_SIZE
REST_LEN = SNAP - OUT_LEN

NC, NS = 2, 16
NW = NC * NS
LANES = 16
G = 3
ZCHUNK = OUT_LEN // NW
NGROUPS = 4
UG = NUM_UPD // NGROUPS
SBLK_LEN = (2 * G - 1) * UPDATE_SIZE

_mesh = plsc.VectorSubcoreMesh(core_axis_name="c", subcore_axis_name="s")


@functools.partial(
    pl.kernel,
    out_type=(
        jax.ShapeDtypeStruct((OUT_LEN,), jnp.float32),
        jax.ShapeDtypeStruct((SNAP,), jnp.float32),
    ),
    mesh=_mesh,
    scratch_types=(
        pltpu.VMEM((UG, G, UPDATE_SIZE), jnp.float32),
        pltpu.VMEM((UG, G, UPDATE_SIZE), jnp.float32),
        pltpu.VMEM((SBLK_LEN,), jnp.float32),
        pltpu.VMEM((G * UPDATE_SIZE,), jnp.float32),
        pltpu.VMEM((ZCHUNK,), jnp.float32),
        pltpu.SemaphoreType.DMA,
        pltpu.SemaphoreType.DMA,
        pltpu.SemaphoreType.DMA,
    ),
)
def _sc_averager_c1(upd_hbm, snap_hbm, out_hbm, newsnap_hbm, buf0, buf1, sblk,
                    res, zbuf, sem0, sem1, sem_o):
    core = lax.axis_index("c")
    sub = lax.axis_index("s")
    w = core * NS + sub
    k0 = w * G

    bufs = (buf0, buf1)
    sems = (sem0, sem1)

    b0 = [jnp.clip(k0 - u, 0, BATCH - G) for u in range(NUM_UPD)]

    def _fire_group(g):
        slot = g % 2
        hs = []
        for i in range(UG):
            u = g * UG + i
            src = upd_hbm.at[pl.ds(b0[u], G), 1, u, :]
            hs.append(pltpu.async_copy(src, bufs[slot].at[i], sems[slot]))
        return hs

    hs0 = _fire_group(0)
    s0 = jnp.minimum(k0, NBLK - G)
    shift = jnp.clip(k0 - s0, 0, G - 1)
    hs1 = _fire_group(1)
    hs1.append(
        pltpu.async_copy(snap_hbm.at[1, pl.ds(s0 * UPDATE_SIZE, G * UPDATE_SIZE)],
                         sblk.at[pl.ds(0, G * UPDATE_SIZE)], sem1)
    )

    @plsc.parallel_loop(0, ZCHUNK // LANES, unroll=4)
    def _zero_body(i):
        zbuf[pl.ds(i * LANES, LANES)] = jnp.zeros((LANES,), jnp.float32)

    h_zero = pltpu.async_copy(
        zbuf, newsnap_hbm.at[pl.ds(REST_LEN + w * ZCHUNK, ZCHUNK)], sem_o
    )

    def _row_weight(r, u):
        b = k0 + r - u
        valid = jnp.logical_and(b >= 0, b <= BATCH - 1)
        row = jnp.clip(b - b0[u], 0, G - 1)
        wt = jnp.where(valid, jnp.float32(1.0 / NUM_UPD), jnp.float32(0.0))
        return row, jnp.broadcast_to(wt, (LANES,))

    def _acc_pass(g, first=False, with_snap=False):
        buf = bufs[g % 2]
        rw = [[_row_weight(r, g * UG + q) for q in range(UG)] for r in range(G)]

        @plsc.parallel_loop(0, UPDATE_SIZE // LANES, unroll=2)
        def _chunk(i):
            o = i * LANES
            for r in range(G):
                if first:
                    acc = buf[0, rw[r][0][0], pl.ds(o, LANES)] * rw[r][0][1]
                    qs = range(1, UG)
                else:
                    acc = res[pl.ds(r * UPDATE_SIZE + o, LANES)]
                    qs = range(UG)
                if with_snap:
                    acc = acc + sblk[pl.ds((r + shift) * UPDATE_SIZE + o, LANES)]
                for q in qs:
                    acc = acc + buf[q, rw[r][q][0], pl.ds(o, LANES)] * rw[r][q][1]
                res[pl.ds(r * UPDATE_SIZE + o, LANES)] = acc

    for h in hs0:
        h.wait()
    _acc_pass(0, first=True)
    hs2 = _fire_group(2)
    for h in hs1:
        h.wait()
    _acc_pass(1, with_snap=True)
    hs3 = _fire_group(3)
    for h in hs2:
        h.wait()
    _acc_pass(2)
    for h in hs3:
        h.wait()
    _acc_pass(3)

    for r in range(G):
        k = k0 + r

        @pl.when(k < BATCH)
        def _(r=r, k=k):
            pltpu.async_copy(
                res.at[pl.ds(r * UPDATE_SIZE, UPDATE_SIZE)],
                out_hbm.at[pl.ds(k * UPDATE_SIZE, UPDATE_SIZE)],
                sem_o,
            )

        @pl.when(jnp.logical_and(k >= BATCH, k < NBLK))
        def _(r=r, k=k):
            pltpu.async_copy(
                res.at[pl.ds(r * UPDATE_SIZE, UPDATE_SIZE)],
                newsnap_hbm.at[pl.ds((k - BATCH) * UPDATE_SIZE, UPDATE_SIZE)],
                sem_o,
            )

    h_zero.wait()
    for r in range(G):
        @pl.when(k0 + r < NBLK)
        def _(r=r):
            pltpu.make_async_copy(
                snap_hbm.at[0, pl.ds(0, UPDATE_SIZE)],
                res.at[pl.ds(r * UPDATE_SIZE, UPDATE_SIZE)],
                sem_o,
            ).wait()


def _tc_body(upd_ref, snap_ref, out_ref, ns_ref, acc_ref):
    x = upd_ref[:, 0]
    acc_ref[...] = jnp.zeros((NBLK, UPDATE_SIZE), jnp.float32)
    for u in range(NUM_UPD):
        acc_ref[u:u + BATCH, :] = acc_ref[u:u + BATCH, :] + x[:, u, :]
    total = snap_ref[0] + acc_ref[...] * jnp.float32(1.0 / NUM_UPD)
    out_ref[...] = total[:BATCH]
    ns_ref[...] = jnp.concatenate(
        [total[BATCH:], jnp.zeros((BATCH, UPDATE_SIZE), jnp.float32)], axis=0
    )


_tc_call = pl.pallas_call(
    _tc_body,
    out_shape=(
        jax.ShapeDtypeStruct((BATCH, UPDATE_SIZE), jnp.float32),
        jax.ShapeDtypeStruct((NBLK, UPDATE_SIZE), jnp.float32),
    ),
    in_specs=[
        pl.BlockSpec((BATCH, 1, NUM_UPD, UPDATE_SIZE), lambda i: (0, 0, 0, 0)),
        pl.BlockSpec((1, NBLK, UPDATE_SIZE), lambda i: (0, 0, 0)),
    ],
    out_specs=(
        pl.BlockSpec((BATCH, UPDATE_SIZE), lambda i: (0, 0)),
        pl.BlockSpec((NBLK, UPDATE_SIZE), lambda i: (0, 0)),
    ),
    scratch_shapes=[pltpu.VMEM((NBLK, UPDATE_SIZE), jnp.float32)],
    grid=(1,),
)


@jax.jit
def kernel(update, snapshot):
    upd_r = update.reshape(BATCH, NUM_CH, NUM_UPD, UPDATE_SIZE)
    snap_r = snapshot.reshape(NUM_CH, NBLK, UPDATE_SIZE)
    out_c1, ns_c1 = _sc_averager_c1(upd_r, snapshot)
    out_c0, ns_c0 = _tc_call(upd_r, snap_r)
    output = jnp.concatenate(
        [out_c0.reshape(1, 1, OUT_LEN), out_c1.reshape(1, 1, OUT_LEN)], axis=1
    )
    new_snapshot = jnp.stack([ns_c0.reshape(SNAP), ns_c1], axis=0)
    return output, new_snapshot


# speedup vs baseline: 1.3961x; 1.3961x over previous
import functools

import jax
import jax.numpy as jnp
from jax import lax
from jax.experimental import pallas as pl
from jax.experimental.pallas import tpu as pltpu
from jax.experimental.pallas import tpu_sc as plsc

UPDATE_SIZE = 1024
BATCH = 64
NUM_UPD = 16
NUM_CH = 2
KEEP = NUM_UPD * UPDATE_SIZE
SNAP = (BATCH + NUM_UPD - 1) * UPDATE_SIZE
NBLK = BATCH + NUM_UPD - 1
OUT_LEN = BATCH * UPDATE_SIZE
REST_LEN = SNAP - OUT_LEN

NC, NS = 2, 16
LANES = 16
G = 5
ZCHUNK = OUT_LEN // NS
NGROUPS = 4
UG = NUM_UPD // NGROUPS

_mesh = plsc.VectorSubcoreMesh(core_axis_name="c", subcore_axis_name="s")


@functools.partial(
    pl.kernel,
    out_type=(
        jax.ShapeDtypeStruct((1, NUM_CH, OUT_LEN), jnp.float32),
        jax.ShapeDtypeStruct((NUM_CH, SNAP), jnp.float32),
    ),
    mesh=_mesh,
    scratch_types=(
        pltpu.VMEM((UG, G, UPDATE_SIZE), jnp.float32),
        pltpu.VMEM((UG, G, UPDATE_SIZE), jnp.float32),
        pltpu.VMEM((G * UPDATE_SIZE,), jnp.float32),
        pltpu.VMEM((G * UPDATE_SIZE,), jnp.float32),
        pltpu.VMEM((ZCHUNK,), jnp.float32),
        pltpu.SemaphoreType.DMA,
        pltpu.SemaphoreType.DMA,
        pltpu.SemaphoreType.DMA,
    ),
)
def _sc_averager(upd_hbm, snap_hbm, out_hbm, newsnap_hbm, buf0, buf1, sblk,
                 res, zbuf, sem0, sem1, sem_o):
    core = lax.axis_index("c")
    sub = lax.axis_index("s")
    w = core * NS + sub
    c = w // NS
    j = w % NS
    k0 = j * G

    bufs = (buf0, buf1)
    sems = (sem0, sem1)

    b0 = [jnp.clip(k0 - u, 0, BATCH - G) for u in range(NUM_UPD)]

    def _fire_group(g):
        slot = g % 2
        hs = []
        for i in range(UG):
            u = g * UG + i
            src = upd_hbm.at[pl.ds(b0[u], G), c, pl.ds(u * UPDATE_SIZE, UPDATE_SIZE)]
            hs.append(pltpu.async_copy(src, bufs[slot].at[i], sems[slot]))
        return hs

    hs0 = _fire_group(0)
    s0 = jnp.minimum(k0, NBLK - G)
    shift = k0 - s0
    hs1 = _fire_group(1)
    hs1.append(
        pltpu.async_copy(snap_hbm.at[c, pl.ds(s0 * UPDATE_SIZE, G * UPDATE_SIZE)],
                         sblk, sem1)
    )

    @plsc.parallel_loop(0, ZCHUNK // LANES, unroll=4)
    def _zero_body(i):
        zbuf[pl.ds(i * LANES, LANES)] = jnp.zeros((LANES,), jnp.float32)

    cz = w % NUM_CH
    jz = w // NUM_CH
    h_zero = pltpu.async_copy(
        zbuf, newsnap_hbm.at[cz, pl.ds(REST_LEN + jz * ZCHUNK, ZCHUNK)], sem_o
    )

    def _row_weight(r, u):
        k = k0 + r
        b = k - u
        valid = jnp.logical_and(b >= 0, b <= BATCH - 1)
        row = jnp.clip(b - b0[u], 0, G - 1)
        wt = jnp.where(valid, jnp.float32(1.0 / NUM_UPD), jnp.float32(0.0))
        return row, jnp.broadcast_to(wt, (LANES,))

    def _acc_pass(g, first=False, with_snap=False):
        slot = g % 2
        buf = bufs[slot]
        rw = [[_row_weight(r, g * UG + q) for q in range(UG)] for r in range(G)]

        @plsc.parallel_loop(0, UPDATE_SIZE // LANES, unroll=2)
        def _chunk(i):
            o = i * LANES
            for r in range(G):
                if first:
                    acc = buf[0, rw[r][0][0], pl.ds(o, LANES)] * rw[r][0][1]
                    qs = range(1, UG)
                else:
                    acc = res[pl.ds(r * UPDATE_SIZE + o, LANES)]
                    qs = range(UG)
                if with_snap:
                    acc = acc + sblk[pl.ds((r + shift) * UPDATE_SIZE + o, LANES)]
                for q in qs:
                    acc = acc + buf[q, rw[r][q][0], pl.ds(o, LANES)] * rw[r][q][1]
                res[pl.ds(r * UPDATE_SIZE + o, LANES)] = acc

    for h in hs0:
        h.wait()
    _acc_pass(0, first=True)
    hs2 = _fire_group(2)
    for h in hs1:
        h.wait()
    _acc_pass(1, with_snap=True)
    hs3 = _fire_group(3)
    for h in hs2:
        h.wait()
    _acc_pass(2)
    for h in hs3:
        h.wait()
    _acc_pass(3)

    @pl.when(k0 + G <= BATCH)
    def _():
        pltpu.async_copy(
            res, out_hbm.at[0, c, pl.ds(k0 * UPDATE_SIZE, G * UPDATE_SIZE)], sem_o
        )

    @pl.when(jnp.logical_and(k0 < BATCH, k0 + G > BATCH))
    def _():
        pltpu.async_copy(
            res.at[pl.ds(0, (G - 1) * UPDATE_SIZE)],
            out_hbm.at[0, c, pl.ds(k0 * UPDATE_SIZE, (G - 1) * UPDATE_SIZE)],
            sem_o,
        )
        pltpu.async_copy(
            res.at[pl.ds((G - 1) * UPDATE_SIZE, UPDATE_SIZE)],
            newsnap_hbm.at[c, pl.ds(0, UPDATE_SIZE)],
            sem_o,
        )

    @pl.when(jnp.logical_and(k0 >= BATCH, k0 + G <= NBLK))
    def _():
        pltpu.async_copy(
            res, newsnap_hbm.at[c, pl.ds((k0 - BATCH) * UPDATE_SIZE, G * UPDATE_SIZE)],
            sem_o,
        )

    @pl.when(k0 + G > NBLK)
    def _():
        pltpu.async_copy(
            res.at[pl.ds(0, (G - 1) * UPDATE_SIZE)],
            newsnap_hbm.at[c, pl.ds((k0 - BATCH) * UPDATE_SIZE, (G - 1) * UPDATE_SIZE)],
            sem_o,
        )

    h_zero.wait()

    @pl.when(k0 + G <= NBLK)
    def _():
        pltpu.make_async_copy(
            snap_hbm.at[0, pl.ds(0, G * UPDATE_SIZE)], res, sem_o
        ).wait()

    @pl.when(k0 + G > NBLK)
    def _():
        pltpu.make_async_copy(
            snap_hbm.at[0, pl.ds(0, (G - 1) * UPDATE_SIZE)],
            res.at[pl.ds(0, (G - 1) * UPDATE_SIZE)],
            sem_o,
        ).wait()


@jax.jit
def kernel(update, snapshot):
    return _sc_averager(update, snapshot)


# speedup vs baseline: 1.4179x; 1.0156x over previous
import functools

import jax
import jax.numpy as jnp
from jax import lax
from jax.experimental import pallas as pl
from jax.experimental.pallas import tpu as pltpu
from jax.experimental.pallas import tpu_sc as plsc

UPDATE_SIZE = 1024
BATCH = 64
NUM_UPD = 16
NUM_CH = 2
KEEP = NUM_UPD * UPDATE_SIZE
SNAP = (BATCH + NUM_UPD - 1) * UPDATE_SIZE
NBLK = BATCH + NUM_UPD - 1
OUT_LEN = BATCH * UPDATE_SIZE
REST_LEN = SNAP - OUT_LEN

NC, NS = 2, 16
LANES = 16
G = 5
ZCHUNK = OUT_LEN // NS
NGROUPS = 4
UG = NUM_UPD // NGROUPS

_mesh = plsc.VectorSubcoreMesh(core_axis_name="c", subcore_axis_name="s")


@functools.partial(
    pl.kernel,
    out_type=(
        jax.ShapeDtypeStruct((1, NUM_CH, OUT_LEN), jnp.float32),
        jax.ShapeDtypeStruct((NUM_CH, SNAP), jnp.float32),
    ),
    mesh=_mesh,
    scratch_types=(
        pltpu.VMEM((UG, G, UPDATE_SIZE), jnp.float32),
        pltpu.VMEM((UG, G, UPDATE_SIZE), jnp.float32),
        pltpu.VMEM((G * UPDATE_SIZE,), jnp.float32),
        pltpu.VMEM((G * UPDATE_SIZE,), jnp.float32),
        pltpu.VMEM((ZCHUNK,), jnp.float32),
        pltpu.SemaphoreType.DMA,
        pltpu.SemaphoreType.DMA,
        pltpu.SemaphoreType.DMA,
    ),
)
def _sc_averager(upd_hbm, snap_hbm, out_hbm, newsnap_hbm, buf0, buf1, sblk,
                 res, zbuf, sem0, sem1, sem_o):
    core = lax.axis_index("c")
    sub = lax.axis_index("s")
    w = core * NS + sub
    c = w // NS
    j = w % NS
    k0 = j * G

    bufs = (buf0, buf1)
    sems = (sem0, sem1)

    b0 = [jnp.clip(k0 - u, 0, BATCH - G) for u in range(NUM_UPD)]

    def _fire_group(g):
        slot = g % 2
        hs = []
        for i in range(UG):
            u = g * UG + i
            src = upd_hbm.at[pl.ds(b0[u], G), c, pl.ds(u * UPDATE_SIZE, UPDATE_SIZE)]
            hs.append(pltpu.async_copy(src, bufs[slot].at[i], sems[slot]))
        return hs

    hs0 = _fire_group(0)
    s0 = jnp.minimum(k0, NBLK - G)
    shift = k0 - s0
    hs1 = _fire_group(1)
    hs1.append(
        pltpu.async_copy(snap_hbm.at[c, pl.ds(s0 * UPDATE_SIZE, G * UPDATE_SIZE)],
                         sblk, sem1)
    )

    @plsc.parallel_loop(0, ZCHUNK // LANES, unroll=1)
    def _zero_body(i):
        zbuf[pl.ds(i * LANES, LANES)] = jnp.zeros((LANES,), jnp.float32)

    cz = w % NUM_CH
    jz = w // NUM_CH
    h_zero = pltpu.async_copy(
        zbuf, newsnap_hbm.at[cz, pl.ds(REST_LEN + jz * ZCHUNK, ZCHUNK)], sem_o
    )

    def _row_weight(r, u):
        k = k0 + r
        b = k - u
        valid = jnp.logical_and(b >= 0, b <= BATCH - 1)
        row = jnp.clip(b - b0[u], 0, G - 1)
        wt = jnp.where(valid, jnp.float32(1.0 / NUM_UPD), jnp.float32(0.0))
        return row, jnp.broadcast_to(wt, (LANES,))

    def _acc_pass(g, first=False, with_snap=False):
        slot = g % 2
        buf = bufs[slot]
        rw = [[_row_weight(r, g * UG + q) for q in range(UG)] for r in range(G)]

        @plsc.parallel_loop(0, UPDATE_SIZE // LANES, unroll=1)
        def _chunk(i):
            o = i * LANES
            for r in range(G):
                if first:
                    acc = buf[0, rw[r][0][0], pl.ds(o, LANES)] * rw[r][0][1]
                    qs = range(1, UG)
                else:
                    acc = res[pl.ds(r * UPDATE_SIZE + o, LANES)]
                    qs = range(UG)
                if with_snap:
                    acc = acc + sblk[pl.ds((r + shift) * UPDATE_SIZE + o, LANES)]
                for q in qs:
                    acc = acc + buf[q, rw[r][q][0], pl.ds(o, LANES)] * rw[r][q][1]
                res[pl.ds(r * UPDATE_SIZE + o, LANES)] = acc

    for h in hs0:
        h.wait()
    _acc_pass(0, first=True)
    hs2 = _fire_group(2)
    for h in hs1:
        h.wait()
    _acc_pass(1, with_snap=True)
    hs3 = _fire_group(3)
    for h in hs2:
        h.wait()
    _acc_pass(2)
    for h in hs3:
        h.wait()
    _acc_pass(3)

    @pl.when(k0 + G <= BATCH)
    def _():
        pltpu.async_copy(
            res, out_hbm.at[0, c, pl.ds(k0 * UPDATE_SIZE, G * UPDATE_SIZE)], sem_o
        )

    @pl.when(jnp.logical_and(k0 < BATCH, k0 + G > BATCH))
    def _():
        pltpu.async_copy(
            res.at[pl.ds(0, (G - 1) * UPDATE_SIZE)],
            out_hbm.at[0, c, pl.ds(k0 * UPDATE_SIZE, (G - 1) * UPDATE_SIZE)],
            sem_o,
        )
        pltpu.async_copy(
            res.at[pl.ds((G - 1) * UPDATE_SIZE, UPDATE_SIZE)],
            newsnap_hbm.at[c, pl.ds(0, UPDATE_SIZE)],
            sem_o,
        )

    @pl.when(jnp.logical_and(k0 >= BATCH, k0 + G <= NBLK))
    def _():
        pltpu.async_copy(
            res, newsnap_hbm.at[c, pl.ds((k0 - BATCH) * UPDATE_SIZE, G * UPDATE_SIZE)],
            sem_o,
        )

    @pl.when(k0 + G > NBLK)
    def _():
        pltpu.async_copy(
            res.at[pl.ds(0, (G - 1) * UPDATE_SIZE)],
            newsnap_hbm.at[c, pl.ds((k0 - BATCH) * UPDATE_SIZE, (G - 1) * UPDATE_SIZE)],
            sem_o,
        )

    h_zero.wait()

    @pl.when(k0 + G <= NBLK)
    def _():
        pltpu.make_async_copy(
            snap_hbm.at[0, pl.ds(0, G * UPDATE_SIZE)], res, sem_o
        ).wait()

    @pl.when(k0 + G > NBLK)
    def _():
        pltpu.make_async_copy(
            snap_hbm.at[0, pl.ds(0, (G - 1) * UPDATE_SIZE)],
            res.at[pl.ds(0, (G - 1) * UPDATE_SIZE)],
            sem_o,
        ).wait()


@jax.jit
def kernel(update, snapshot):
    return _sc_averager(update, snapshot)
